# baseline (speedup 1.0000x reference)
import functools

import jax
import jax.numpy as jnp
from jax import lax
from jax.experimental import pallas as pl
from jax.experimental.pallas import tpu as pltpu
from jax.experimental.pallas import tpu_sc as plsc

BATCH = 16384
D = 64
N_USERS = 1000001
N_ITEMS = 1000000
NSEM = 8

_info = plsc.get_sparse_core_info()
NC, NS = _info.num_cores, _info.num_subcores
NW = NC * NS
B_PER_W = BATCH // NW
CH = 256
N_CHUNKS = B_PER_W // CH


def _make_gather_kernel():
    mesh = plsc.VectorSubcoreMesh(core_axis_name="c", subcore_axis_name="s")

    @functools.partial(
        pl.kernel,
        mesh=mesh,
        out_type=(
            jax.ShapeDtypeStruct((BATCH, D), jnp.float32),
            jax.ShapeDtypeStruct((BATCH, D), jnp.float32),
        ),
        scratch_types=[
            pltpu.VMEM((B_PER_W,), jnp.int32),
            pltpu.VMEM((CH, D), jnp.float32),
            pltpu.VMEM((B_PER_W,), jnp.int32),
            pltpu.VMEM((CH, D), jnp.float32),
            [pltpu.SemaphoreType.DMA] * NSEM,
            [pltpu.SemaphoreType.DMA] * NSEM,
        ],
    )
    def gather_kernel(
        user_hbm,
        item_hbm,
        uw_hbm,
        iw_hbm,
        uout_hbm,
        iout_hbm,
        uidx_v,
        urows_v,
        iidx_v,
        irows_v,
        usems,
        isems,
    ):
        wid = lax.axis_index("s") * NC + lax.axis_index("c")
        base = wid * B_PER_W
        pltpu.sync_copy(user_hbm.at[pl.ds(base, B_PER_W)], uidx_v)
        pltpu.sync_copy(item_hbm.at[pl.ds(base, B_PER_W)], iidx_v)

        for c in range(N_CHUNKS):
            off = c * CH

            def issue_u(g, _):
                v = uidx_v[pl.ds(off + g * 16, 16)]
                for k in range(16):
                    pltpu.async_copy(
                        uw_hbm.at[pl.ds(v[k], 1), :],
                        urows_v.at[pl.ds(g * 16 + k, 1), :],
                        usems[k % NSEM],
                    )
                return 0

            def issue_i(g, _):
                v = iidx_v[pl.ds(off + g * 16, 16)]
                for k in range(16):
                    pltpu.async_copy(
                        iw_hbm.at[pl.ds(v[k], 1), :],
                        irows_v.at[pl.ds(g * 16 + k, 1), :],
                        isems[k % NSEM],
                    )
                return 0

            lax.fori_loop(0, CH // 16, issue_u, 0)
            lax.fori_loop(0, CH // 16, issue_i, 0)

            def drain_u(g, _):
                for k in range(16):
                    pltpu.make_async_copy(
                        uw_hbm.at[pl.ds(0, 1), :],
                        urows_v.at[pl.ds(g * 16 + k, 1), :],
                        usems[k % NSEM],
                    ).wait()
                return 0

            def drain_i(g, _):
                for k in range(16):
                    pltpu.make_async_copy(
                        iw_hbm.at[pl.ds(0, 1), :],
                        irows_v.at[pl.ds(g * 16 + k, 1), :],
                        isems[k % NSEM],
                    ).wait()
                return 0

            lax.fori_loop(0, CH // 16, drain_u, 0)
            pltpu.sync_copy(urows_v, uout_hbm.at[pl.ds(base + off, CH)])
            lax.fori_loop(0, CH // 16, drain_i, 0)
            pltpu.sync_copy(irows_v, iout_hbm.at[pl.ds(base + off, CH)])

    return gather_kernel


_gather = _make_gather_kernel()


@jax.jit
def kernel(user, item, user_weight, item_weight):
    user = user.astype(jnp.int32)
    item = item.astype(jnp.int32)
    return _gather(user, item, user_weight, item_weight)
